# baseline (speedup 1.0000x reference)
import functools

import jax
import jax.numpy as jnp
from jax import lax
from jax.experimental import pallas as pl
from jax.experimental.pallas import tpu as pltpu

N = 10000
D = 128
BLK = 2000


def _mid_kernel(a_ref, x_ref, wrel1_ref, brel1_ref, wroot1_ref,
                wrel2_ref, wroot2_ref, wfc_ref, g_ref, t_ref):
    a = a_ref[...]
    x = x_ref[...]
    h = jnp.dot(a, wrel1_ref[...].T, preferred_element_type=jnp.float32)
    h = h + jnp.dot(x, wroot1_ref[...].T, preferred_element_type=jnp.float32)
    h = jnp.maximum(h + brel1_ref[...], 0.0)
    v1 = jnp.dot(wfc_ref[...], wrel2_ref[...], preferred_element_type=jnp.float32)
    v2 = jnp.dot(wfc_ref[...], wroot2_ref[...], preferred_element_type=jnp.float32)
    g_ref[...] = jnp.broadcast_to(jnp.sum(h * v1, axis=1, keepdims=True), h.shape)
    t_ref[...] = jnp.broadcast_to(jnp.sum(h * v2, axis=1, keepdims=True), h.shape)


def _mid_stage(agg, x, Wrel1, brel1, Wroot1, Wrel2, Wroot2, Wfc):
    full = pl.BlockSpec((D, D), lambda i: (0, 0))
    g, t = pl.pallas_call(
        _mid_kernel,
        grid=(N // BLK,),
        in_specs=[
            pl.BlockSpec((BLK, D), lambda i: (i, 0)),
            pl.BlockSpec((BLK, D), lambda i: (i, 0)),
            full,
            pl.BlockSpec((1, D), lambda i: (0, 0)),
            full, full, full,
            pl.BlockSpec((1, D), lambda i: (0, 0)),
        ],
        out_specs=[pl.BlockSpec((BLK, D), lambda i: (i, 0)),
                   pl.BlockSpec((BLK, D), lambda i: (i, 0))],
        out_shape=[jax.ShapeDtypeStruct((N, D), jnp.float32),
                   jax.ShapeDtypeStruct((N, D), jnp.float32)],
    )(agg, x, Wrel1, brel1.reshape(1, D), Wroot1, Wrel2, Wroot2, Wfc)
    return g[:, 0], t[:, 0]


def kernel(x, edge_index, Wrel1, brel1, Wroot1, Wrel2, brel2, Wroot2, Wfc, bfc):
    src = edge_index[0].astype(jnp.int32)
    dst = edge_index[1].astype(jnp.int32)
    agg = jax.ops.segment_sum(jnp.take(x, src, axis=0), dst, num_segments=N)
    g, t = _mid_stage(agg, x, Wrel1, brel1, Wroot1, Wrel2, Wroot2, Wfc)
    s = jax.ops.segment_sum(jnp.take(g, src, axis=0), dst, num_segments=N)
    c = jnp.dot(brel2, Wfc[0]) + bfc[0]
    return s + t + c


# speedup vs baseline: 10.1189x; 10.1189x over previous
import functools

import jax
import jax.numpy as jnp
from jax import lax
from jax.experimental import pallas as pl
from jax.experimental.pallas import tpu as pltpu
from jax.experimental.pallas import tpu_sc as plsc

N = 10000
E = 320000
D = 128

NC = 2
NS = 16
NW = NC * NS

NP = 10240
RPT = NP // NS
EW = E // NW
CH = 80
NCH = EW // CH
NCHP = 128

_mesh = plsc.VectorSubcoreMesh(core_axis_name="c", subcore_axis_name="s")


def _seg_rows_body(x_hbm, src_hbm, dst_hbm, zeros_hbm, out0_hbm, out1_hbm,
                   src_v, dst_v, rows_v, acc_sh, sem):
    cid = lax.axis_index("c")
    sid = lax.axis_index("s")
    wid = cid * NS + sid
    pltpu.sync_copy(zeros_hbm.at[pl.ds(sid * RPT, RPT)],
                    acc_sh.at[pl.ds(sid * RPT, RPT)])
    pltpu.sync_copy(src_hbm.at[pl.ds(wid * NCHP, NCHP)], src_v)
    pltpu.sync_copy(dst_hbm.at[pl.ds(wid * NCHP, NCHP)], dst_v)
    plsc.subcore_barrier()

    def step(j, carry):
        pltpu.async_copy(x_hbm.at[src_v.at[j]], rows_v, sem).wait()
        pltpu.sync_copy(rows_v, acc_sh.at[dst_v.at[j]], add=True)
        return carry

    lax.fori_loop(0, NCH, step, 0)
    plsc.subcore_barrier()

    @pl.when(cid == 0)
    def _():
        pltpu.sync_copy(acc_sh.at[pl.ds(sid * RPT, RPT)],
                        out0_hbm.at[pl.ds(sid * RPT, RPT)])

    @pl.when(cid == 1)
    def _():
        pltpu.sync_copy(acc_sh.at[pl.ds(sid * RPT, RPT)],
                        out1_hbm.at[pl.ds(sid * RPT, RPT)])


_seg_rows = functools.partial(
    pl.kernel,
    out_type=[jax.ShapeDtypeStruct((NP, D), jnp.float32),
              jax.ShapeDtypeStruct((NP, D), jnp.float32)],
    mesh=_mesh,
    scratch_types=[
        pltpu.VMEM((NCHP, CH), jnp.int32),
        pltpu.VMEM((NCHP, CH), jnp.int32),
        pltpu.VMEM((CH, D), jnp.float32),
        pltpu.VMEM_SHARED((NP, D), jnp.float32),
        pltpu.SemaphoreType.DMA,
    ],
)(_seg_rows_body)


DG = 16


def _seg_scalar_body(g_hbm, src_hbm, dst_hbm, zeros_hbm, out0_hbm, out1_hbm,
                     src_v, dst_v, vals_v, acc_sh, sem):
    cid = lax.axis_index("c")
    sid = lax.axis_index("s")
    wid = cid * NS + sid
    pltpu.sync_copy(zeros_hbm.at[pl.ds(sid * RPT, RPT)],
                    acc_sh.at[pl.ds(sid * RPT, RPT)])
    pltpu.sync_copy(src_hbm.at[pl.ds(wid * NCHP, NCHP)], src_v)
    pltpu.sync_copy(dst_hbm.at[pl.ds(wid * NCHP, NCHP)], dst_v)
    plsc.subcore_barrier()

    def step(j, carry):
        pltpu.async_copy(g_hbm.at[src_v.at[j]], vals_v, sem).wait()
        pltpu.sync_copy(vals_v, acc_sh.at[dst_v.at[j]], add=True)
        return carry

    lax.fori_loop(0, NCH, step, 0)
    plsc.subcore_barrier()

    @pl.when(cid == 0)
    def _():
        pltpu.sync_copy(acc_sh.at[pl.ds(sid * RPT, RPT)],
                        out0_hbm.at[pl.ds(sid * RPT, RPT)])

    @pl.when(cid == 1)
    def _():
        pltpu.sync_copy(acc_sh.at[pl.ds(sid * RPT, RPT)],
                        out1_hbm.at[pl.ds(sid * RPT, RPT)])


_seg_scalar = functools.partial(
    pl.kernel,
    out_type=[jax.ShapeDtypeStruct((NP, DG), jnp.float32),
              jax.ShapeDtypeStruct((NP, DG), jnp.float32)],
    mesh=_mesh,
    compiler_params=pltpu.CompilerParams(use_tc_tiling_on_sc=False),
    scratch_types=[
        pltpu.VMEM((NCHP, CH), jnp.int32),
        pltpu.VMEM((NCHP, CH), jnp.int32),
        pltpu.VMEM((CH, DG), jnp.float32),
        pltpu.VMEM_SHARED((NP, DG), jnp.float32),
        pltpu.SemaphoreType.DMA,
    ],
)(_seg_scalar_body)


BLK = 2000


def _mid_kernel(a0_ref, a1_ref, x_ref, wrel1_ref, brel1_ref, wroot1_ref,
                wrel2_ref, wroot2_ref, wfc_ref, g_ref, t_ref):
    a = a0_ref[...] + a1_ref[...]
    x = x_ref[...]
    h = jnp.dot(a, wrel1_ref[...].T, preferred_element_type=jnp.float32)
    h = h + jnp.dot(x, wroot1_ref[...].T, preferred_element_type=jnp.float32)
    h = jnp.maximum(h + brel1_ref[...], 0.0)
    v1 = jnp.dot(wfc_ref[...], wrel2_ref[...], preferred_element_type=jnp.float32)
    v2 = jnp.dot(wfc_ref[...], wroot2_ref[...], preferred_element_type=jnp.float32)
    g_ref[...] = jnp.broadcast_to(jnp.sum(h * v1, axis=1, keepdims=True), h.shape)
    t_ref[...] = jnp.broadcast_to(jnp.sum(h * v2, axis=1, keepdims=True), h.shape)


def _mid_stage(s0, s1, x, Wrel1, brel1, Wroot1, Wrel2, Wroot2, Wfc):
    full = pl.BlockSpec((D, D), lambda i: (0, 0))
    row1 = pl.BlockSpec((1, D), lambda i: (0, 0))
    blk = pl.BlockSpec((BLK, D), lambda i: (i, 0))
    g, t = pl.pallas_call(
        _mid_kernel,
        grid=(N // BLK,),
        in_specs=[blk, blk, blk, full, row1, full, full, full, row1],
        out_specs=[blk, blk],
        out_shape=[jax.ShapeDtypeStruct((N, D), jnp.float32),
                   jax.ShapeDtypeStruct((N, D), jnp.float32)],
    )(s0, s1, x, Wrel1, brel1.reshape(1, D), Wroot1, Wrel2, Wroot2, Wfc)
    return g[:, :DG], t[:, 0]


FR = NP // D


def _final_kernel(s0_ref, s1_ref, t_ref, brel2_ref, wfc_ref, bfc_ref, o_ref):
    c = jnp.sum(brel2_ref[...] * wfc_ref[...]) + bfc_ref[0, 0]
    o_ref[...] = s0_ref[...] + s1_ref[...] + t_ref[...] + c


def _final_stage(s0, s1, t_p, brel2, Wfc, bfc):
    fullb = pl.BlockSpec((FR, D), lambda: (0, 0))
    row1 = pl.BlockSpec((1, D), lambda: (0, 0))
    out = pl.pallas_call(
        _final_kernel,
        in_specs=[fullb, fullb, fullb, row1, row1,
                  pl.BlockSpec((1, 1), lambda: (0, 0))],
        out_specs=fullb,
        out_shape=jax.ShapeDtypeStruct((FR, D), jnp.float32),
    )(s0.reshape(FR, D), s1.reshape(FR, D), t_p.reshape(FR, D),
      brel2.reshape(1, D), Wfc, bfc.reshape(1, 1))
    return out.reshape(NP)[:N]


def kernel(x, edge_index, Wrel1, brel1, Wroot1, Wrel2, brel2, Wroot2, Wfc, bfc):
    src = edge_index[0].astype(jnp.int32)
    dst = edge_index[1].astype(jnp.int32)
    src2 = jnp.pad(src.reshape(NW, NCH, CH), ((0, 0), (0, NCHP - NCH), (0, 0))
                   ).reshape(NW * NCHP, CH)
    dst2 = jnp.pad(dst.reshape(NW, NCH, CH), ((0, 0), (0, NCHP - NCH), (0, 0))
                   ).reshape(NW * NCHP, CH)
    zeros2d = jnp.zeros((NP, D), jnp.float32)

    s0, s1 = _seg_rows(x, src2, dst2, zeros2d)
    g16, t = _mid_stage(s0[:N], s1[:N], x, Wrel1, brel1, Wroot1,
                        Wrel2, Wroot2, Wfc)
    z0, z1 = _seg_scalar(g16, src2, dst2, jnp.zeros((NP, DG), jnp.float32))
    t_p = jnp.concatenate([t, jnp.zeros((NP - N,), jnp.float32)])
    return _final_stage(z0[:, 0], z1[:, 0], t_p, brel2, Wfc, bfc)


# speedup vs baseline: 15.0787x; 1.4901x over previous
import functools

import jax
import jax.numpy as jnp
from jax import lax
from jax.experimental import pallas as pl
from jax.experimental.pallas import tpu as pltpu
from jax.experimental.pallas import tpu_sc as plsc

N = 10000
E = 320000
D = 128

NC = 2
NS = 16
NW = NC * NS

NP = 10240
RPT = NP // NS
EW = E // NW
CH = 80
NCH = EW // CH
NCHP = 128

_mesh = plsc.VectorSubcoreMesh(core_axis_name="c", subcore_axis_name="s")


DG = 16
NBUF = 2
LOOK = 1


def _make_seg_body(W):
    R = 25

    def body(tab_hbm, src_hbm, dst_hbm, zeros_hbm, out0_hbm, out1_hbm,
             src_v, dst_v, rows_v, acc_sh, *sems):
        gs, ss = sems[:NBUF], sems[NBUF:]
        cid = lax.axis_index("c")
        sid = lax.axis_index("s")
        wid = cid * NS + sid

        pltpu.sync_copy(zeros_hbm.at[pl.ds(sid * RPT, RPT)],
                        acc_sh.at[pl.ds(sid * RPT, RPT)])
        pltpu.sync_copy(src_hbm.at[pl.ds(wid * NCHP, NCHP)], src_v)
        pltpu.sync_copy(dst_hbm.at[pl.ds(wid * NCHP, NCHP)], dst_v)
        plsc.subcore_barrier()

        def round_body(it, carry):
            base = it * R

            def gather(s):
                b = s % NBUF
                return pltpu.async_copy(tab_hbm.at[src_v.at[base + s]],
                                        rows_v.at[b], gs[b])

            def scatter(s):
                b = s % NBUF
                return pltpu.async_copy(rows_v.at[b],
                                        acc_sh.at[dst_v.at[base + s]],
                                        ss[b], add=True)

            gd = [gather(s) for s in range(LOOK)] + [None] * (R - LOOK)
            sd = [None] * R
            for s in range(R):
                if s >= NBUF - LOOK:
                    sd[s - (NBUF - LOOK)].wait()
                if s + LOOK < R:
                    gd[s + LOOK] = gather(s + LOOK)
                gd[s].wait()
                sd[s] = scatter(s)
            for s in range(R - (NBUF - LOOK), R):
                sd[s].wait()
            return carry

        lax.fori_loop(0, NCH // R, round_body, 0, unroll=False)
        plsc.subcore_barrier()

        @pl.when(cid == 0)
        def _():
            pltpu.sync_copy(acc_sh.at[pl.ds(sid * RPT, RPT)],
                            out0_hbm.at[pl.ds(sid * RPT, RPT)])

        @pl.when(cid == 1)
        def _():
            pltpu.sync_copy(acc_sh.at[pl.ds(sid * RPT, RPT)],
                            out1_hbm.at[pl.ds(sid * RPT, RPT)])

    return body


def _make_seg_kernel(W, tc_tiling):
    return functools.partial(
        pl.kernel,
        out_type=[jax.ShapeDtypeStruct((NP, W), jnp.float32),
                  jax.ShapeDtypeStruct((NP, W), jnp.float32)],
        mesh=_mesh,
        compiler_params=pltpu.CompilerParams(use_tc_tiling_on_sc=tc_tiling),
        scratch_types=[
            pltpu.VMEM((NCHP, CH), jnp.int32),
            pltpu.VMEM((NCHP, CH), jnp.int32),
            pltpu.VMEM((NBUF, CH, W), jnp.float32),
            pltpu.VMEM_SHARED((NP, W), jnp.float32),
        ] + [pltpu.SemaphoreType.DMA] * (2 * NBUF),
    )(_make_seg_body(W))


_seg_rows = _make_seg_kernel(D, False)
_seg_scalar = _make_seg_kernel(DG, False)


BLK = 2000


def _mid_kernel(a0_ref, a1_ref, x_ref, wrel1_ref, brel1_ref, wroot1_ref,
                wrel2_ref, wroot2_ref, wfc_ref, g_ref, t_ref):
    a = a0_ref[...] + a1_ref[...]
    x = x_ref[...]
    h = jnp.dot(a, wrel1_ref[...].T, preferred_element_type=jnp.float32)
    h = h + jnp.dot(x, wroot1_ref[...].T, preferred_element_type=jnp.float32)
    h = jnp.maximum(h + brel1_ref[...], 0.0)
    v1 = jnp.dot(wfc_ref[...], wrel2_ref[...], preferred_element_type=jnp.float32)
    v2 = jnp.dot(wfc_ref[...], wroot2_ref[...], preferred_element_type=jnp.float32)
    g_ref[...] = jnp.broadcast_to(jnp.sum(h * v1, axis=1, keepdims=True), h.shape)
    t_ref[...] = jnp.broadcast_to(jnp.sum(h * v2, axis=1, keepdims=True), h.shape)


def _mid_stage(s0, s1, x, Wrel1, brel1, Wroot1, Wrel2, Wroot2, Wfc):
    full = pl.BlockSpec((D, D), lambda i: (0, 0))
    row1 = pl.BlockSpec((1, D), lambda i: (0, 0))
    blk = pl.BlockSpec((BLK, D), lambda i: (i, 0))
    g, t = pl.pallas_call(
        _mid_kernel,
        grid=(N // BLK,),
        in_specs=[blk, blk, blk, full, row1, full, full, full, row1],
        out_specs=[blk, blk],
        out_shape=[jax.ShapeDtypeStruct((N, D), jnp.float32),
                   jax.ShapeDtypeStruct((N, D), jnp.float32)],
    )(s0, s1, x, Wrel1, brel1.reshape(1, D), Wroot1, Wrel2, Wroot2, Wfc)
    return g[:, :DG], t[:, 0]


FR = NP // D


def _final_kernel(s0_ref, s1_ref, t_ref, brel2_ref, wfc_ref, bfc_ref, o_ref):
    c = jnp.sum(brel2_ref[...] * wfc_ref[...]) + bfc_ref[0, 0]
    o_ref[...] = s0_ref[...] + s1_ref[...] + t_ref[...] + c


def _final_stage(s0, s1, t_p, brel2, Wfc, bfc):
    fullb = pl.BlockSpec((FR, D), lambda: (0, 0))
    row1 = pl.BlockSpec((1, D), lambda: (0, 0))
    out = pl.pallas_call(
        _final_kernel,
        in_specs=[fullb, fullb, fullb, row1, row1,
                  pl.BlockSpec((1, 1), lambda: (0, 0))],
        out_specs=fullb,
        out_shape=jax.ShapeDtypeStruct((FR, D), jnp.float32),
    )(s0.reshape(FR, D), s1.reshape(FR, D), t_p.reshape(FR, D),
      brel2.reshape(1, D), Wfc, bfc.reshape(1, 1))
    return out.reshape(NP)[:N]


def kernel(x, edge_index, Wrel1, brel1, Wroot1, Wrel2, brel2, Wroot2, Wfc, bfc):
    src = edge_index[0].astype(jnp.int32)
    dst = edge_index[1].astype(jnp.int32)
    src2 = jnp.pad(src.reshape(NW, NCH, CH), ((0, 0), (0, NCHP - NCH), (0, 0))
                   ).reshape(NW * NCHP, CH)
    dst2 = jnp.pad(dst.reshape(NW, NCH, CH), ((0, 0), (0, NCHP - NCH), (0, 0))
                   ).reshape(NW * NCHP, CH)
    zeros2d = jnp.zeros((NP, D), jnp.float32)

    s0, s1 = _seg_rows(x, src2, dst2, zeros2d)
    g16, t = _mid_stage(s0[:N], s1[:N], x, Wrel1, brel1, Wroot1,
                        Wrel2, Wroot2, Wfc)
    z0, z1 = _seg_scalar(g16, src2, dst2, jnp.zeros((NP, DG), jnp.float32))
    t_p = jnp.concatenate([t, jnp.zeros((NP - N,), jnp.float32)])
    return _final_stage(z0[:, 0], z1[:, 0], t_p, brel2, Wfc, bfc)


# speedup vs baseline: 16.3498x; 1.0843x over previous
import functools

import jax
import jax.numpy as jnp
from jax import lax
from jax.experimental import pallas as pl
from jax.experimental.pallas import tpu as pltpu
from jax.experimental.pallas import tpu_sc as plsc

N = 10000
E = 320000
D = 128

NC = 2
NS = 16
NW = NC * NS

NP = 10240
RPT = NP // NS
EW = E // NW
CH = 40
NCH = EW // CH
NCHP = NCH

_mesh = plsc.VectorSubcoreMesh(core_axis_name="c", subcore_axis_name="s")


DG = 16


def _make_seg_body(W, NBUF, LOOK):
    R = 25

    def body(tab_hbm, src_hbm, dst_hbm, zeros_hbm, out0_hbm, out1_hbm,
             src_v, dst_v, rows_v, acc_sh, *sems):
        gs, ss = sems[:NBUF], sems[NBUF:]
        cid = lax.axis_index("c")
        sid = lax.axis_index("s")
        wid = cid * NS + sid

        pltpu.sync_copy(zeros_hbm.at[pl.ds(sid * RPT, RPT)],
                        acc_sh.at[pl.ds(sid * RPT, RPT)])
        pltpu.sync_copy(src_hbm.at[pl.ds(wid * NCHP, NCHP)], src_v)
        pltpu.sync_copy(dst_hbm.at[pl.ds(wid * NCHP, NCHP)], dst_v)
        plsc.subcore_barrier()

        def round_body(it, carry):
            base = it * R

            def gather(s):
                b = s % NBUF
                return pltpu.async_copy(tab_hbm.at[src_v.at[base + s]],
                                        rows_v.at[b], gs[b])

            def scatter(s):
                b = s % NBUF
                return pltpu.async_copy(rows_v.at[b],
                                        acc_sh.at[dst_v.at[base + s]],
                                        ss[b], add=True)

            gd = [gather(s) for s in range(LOOK)] + [None] * (R - LOOK)
            sd = [None] * R
            for s in range(R):
                if s >= NBUF - LOOK:
                    sd[s - (NBUF - LOOK)].wait()
                if s + LOOK < R:
                    gd[s + LOOK] = gather(s + LOOK)
                gd[s].wait()
                sd[s] = scatter(s)
            for s in range(R - (NBUF - LOOK), R):
                sd[s].wait()
            return carry

        lax.fori_loop(0, NCH // R, round_body, 0, unroll=False)
        plsc.subcore_barrier()

        @pl.when(cid == 0)
        def _():
            pltpu.sync_copy(acc_sh.at[pl.ds(sid * RPT, RPT)],
                            out0_hbm.at[pl.ds(sid * RPT, RPT)])

        @pl.when(cid == 1)
        def _():
            pltpu.sync_copy(acc_sh.at[pl.ds(sid * RPT, RPT)],
                            out1_hbm.at[pl.ds(sid * RPT, RPT)])

    return body


def _make_seg_kernel(W, tc_tiling, NBUF, LOOK):
    return functools.partial(
        pl.kernel,
        out_type=[jax.ShapeDtypeStruct((NP, W), jnp.float32),
                  jax.ShapeDtypeStruct((NP, W), jnp.float32)],
        mesh=_mesh,
        compiler_params=pltpu.CompilerParams(use_tc_tiling_on_sc=tc_tiling),
        scratch_types=[
            pltpu.VMEM((NCHP, CH), jnp.int32),
            pltpu.VMEM((NCHP, CH), jnp.int32),
            pltpu.VMEM((NBUF, CH, W), jnp.float32),
            pltpu.VMEM_SHARED((NP, W), jnp.float32),
        ] + [pltpu.SemaphoreType.DMA] * (2 * NBUF),
    )(_make_seg_body(W, NBUF, LOOK))


_seg_rows = _make_seg_kernel(D, False, 4, 2)
_seg_scalar = _make_seg_kernel(DG, False, 6, 3)


BLK = 2000


def _mid_kernel(a0_ref, a1_ref, x_ref, wrel1_ref, brel1_ref, wroot1_ref,
                wrel2_ref, wroot2_ref, wfc_ref, g_ref, t_ref):
    a = a0_ref[...] + a1_ref[...]
    x = x_ref[...]
    h = jnp.dot(a, wrel1_ref[...].T, preferred_element_type=jnp.float32)
    h = h + jnp.dot(x, wroot1_ref[...].T, preferred_element_type=jnp.float32)
    h = jnp.maximum(h + brel1_ref[...], 0.0)
    v1 = jnp.dot(wfc_ref[...], wrel2_ref[...], preferred_element_type=jnp.float32)
    v2 = jnp.dot(wfc_ref[...], wroot2_ref[...], preferred_element_type=jnp.float32)
    g_ref[...] = jnp.broadcast_to(jnp.sum(h * v1, axis=1, keepdims=True), h.shape)
    t_ref[...] = jnp.broadcast_to(jnp.sum(h * v2, axis=1, keepdims=True), h.shape)


def _mid_stage(s0, s1, x, Wrel1, brel1, Wroot1, Wrel2, Wroot2, Wfc):
    full = pl.BlockSpec((D, D), lambda i: (0, 0))
    row1 = pl.BlockSpec((1, D), lambda i: (0, 0))
    blk = pl.BlockSpec((BLK, D), lambda i: (i, 0))
    g, t = pl.pallas_call(
        _mid_kernel,
        grid=(N // BLK,),
        in_specs=[blk, blk, blk, full, row1, full, full, full, row1],
        out_specs=[blk, blk],
        out_shape=[jax.ShapeDtypeStruct((N, D), jnp.float32),
                   jax.ShapeDtypeStruct((N, D), jnp.float32)],
    )(s0, s1, x, Wrel1, brel1.reshape(1, D), Wroot1, Wrel2, Wroot2, Wfc)
    return g[:, :DG], t[:, 0]


FR = NP // D


def _final_kernel(s0_ref, s1_ref, t_ref, brel2_ref, wfc_ref, bfc_ref, o_ref):
    c = jnp.sum(brel2_ref[...] * wfc_ref[...]) + bfc_ref[0, 0]
    o_ref[...] = s0_ref[...] + s1_ref[...] + t_ref[...] + c


def _final_stage(s0, s1, t_p, brel2, Wfc, bfc):
    fullb = pl.BlockSpec((FR, D), lambda: (0, 0))
    row1 = pl.BlockSpec((1, D), lambda: (0, 0))
    out = pl.pallas_call(
        _final_kernel,
        in_specs=[fullb, fullb, fullb, row1, row1,
                  pl.BlockSpec((1, 1), lambda: (0, 0))],
        out_specs=fullb,
        out_shape=jax.ShapeDtypeStruct((FR, D), jnp.float32),
    )(s0.reshape(FR, D), s1.reshape(FR, D), t_p.reshape(FR, D),
      brel2.reshape(1, D), Wfc, bfc.reshape(1, 1))
    return out.reshape(NP)[:N]


def kernel(x, edge_index, Wrel1, brel1, Wroot1, Wrel2, brel2, Wroot2, Wfc, bfc):
    src = edge_index[0].astype(jnp.int32)
    dst = edge_index[1].astype(jnp.int32)
    src2 = src.reshape(NW * NCH, CH)
    dst2 = dst.reshape(NW * NCH, CH)
    zeros2d = jnp.zeros((NP, D), jnp.float32)

    s0, s1 = _seg_rows(x, src2, dst2, zeros2d)
    g16, t = _mid_stage(s0[:N], s1[:N], x, Wrel1, brel1, Wroot1,
                        Wrel2, Wroot2, Wfc)
    z0, z1 = _seg_scalar(g16, src2, dst2, jnp.zeros((NP, DG), jnp.float32))
    t_p = jnp.concatenate([t, jnp.zeros((NP - N,), jnp.float32)])
    return _final_stage(z0[:, 0], z1[:, 0], t_p, brel2, Wfc, bfc)


# speedup vs baseline: 17.7077x; 1.0831x over previous
import functools

import jax
import jax.numpy as jnp
from jax import lax
from jax.experimental import pallas as pl
from jax.experimental.pallas import tpu as pltpu
from jax.experimental.pallas import tpu_sc as plsc

N = 10000
E = 320000
D = 128
DG = 16

NC = 2
NS = 16
NW = NC * NS

NP = 10240
RPT = NP // NS
EW = E // NW

_mesh = plsc.VectorSubcoreMesh(core_axis_name="c", subcore_axis_name="s")


def _make_seg_body(W, CH, NBUF, LOOK, R):
    NCH = EW // CH

    def body(tab_hbm, src_hbm, dst_hbm, zeros_hbm, out0_hbm, out1_hbm,
             src_v, dst_v, rows_v, acc_sh, *sems):
        gs, ss = sems[:NBUF], sems[NBUF:]
        cid = lax.axis_index("c")
        sid = lax.axis_index("s")
        wid = cid * NS + sid

        pltpu.sync_copy(zeros_hbm.at[pl.ds(sid * RPT, RPT)],
                        acc_sh.at[pl.ds(sid * RPT, RPT)])
        pltpu.sync_copy(src_hbm.at[pl.ds(wid * NCH, NCH)], src_v)
        pltpu.sync_copy(dst_hbm.at[pl.ds(wid * NCH, NCH)], dst_v)
        plsc.subcore_barrier()

        def round_body(it, carry):
            base = it * R

            def gather(s):
                b = s % NBUF
                return pltpu.async_copy(tab_hbm.at[src_v.at[base + s]],
                                        rows_v.at[b], gs[b])

            def scatter(s):
                b = s % NBUF
                return pltpu.async_copy(rows_v.at[b],
                                        acc_sh.at[dst_v.at[base + s]],
                                        ss[b], add=True)

            gd = [gather(s) for s in range(LOOK)] + [None] * (R - LOOK)
            sd = [None] * R
            for s in range(R):
                if s >= NBUF - LOOK:
                    sd[s - (NBUF - LOOK)].wait()
                if s + LOOK < R:
                    gd[s + LOOK] = gather(s + LOOK)
                gd[s].wait()
                sd[s] = scatter(s)
            for s in range(R - (NBUF - LOOK), R):
                sd[s].wait()
            return carry

        lax.fori_loop(0, NCH // R, round_body, 0, unroll=False)
        plsc.subcore_barrier()

        @pl.when(cid == 0)
        def _():
            pltpu.sync_copy(acc_sh.at[pl.ds(sid * RPT, RPT)],
                            out0_hbm.at[pl.ds(sid * RPT, RPT)])

        @pl.when(cid == 1)
        def _():
            pltpu.sync_copy(acc_sh.at[pl.ds(sid * RPT, RPT)],
                            out1_hbm.at[pl.ds(sid * RPT, RPT)])

    return body


def _make_seg_kernel(W, CH, NBUF, LOOK, R):
    NCH = EW // CH
    return functools.partial(
        pl.kernel,
        out_type=[jax.ShapeDtypeStruct((NP, W), jnp.float32),
                  jax.ShapeDtypeStruct((NP, W), jnp.float32)],
        mesh=_mesh,
        compiler_params=pltpu.CompilerParams(use_tc_tiling_on_sc=False),
        scratch_types=[
            pltpu.VMEM((NCH, CH), jnp.int32),
            pltpu.VMEM((NCH, CH), jnp.int32),
            pltpu.VMEM((NBUF, CH, W), jnp.float32),
            pltpu.VMEM_SHARED((NP, W), jnp.float32),
        ] + [pltpu.SemaphoreType.DMA] * (2 * NBUF),
    )(_make_seg_body(W, CH, NBUF, LOOK, R))


CH1 = 40
CH3 = 125

_seg_rows = _make_seg_kernel(D, CH1, 4, 2, 25)
_seg_scalar = _make_seg_kernel(DG, CH3, 8, 4, 20)


BLK = 2000


def _mid_kernel(a0_ref, a1_ref, x_ref, wrel1_ref, brel1_ref, wroot1_ref,
                wrel2_ref, wroot2_ref, wfc_ref, g_ref, t_ref):
    a = a0_ref[...] + a1_ref[...]
    x = x_ref[...]
    h = jnp.dot(a, wrel1_ref[...].T, preferred_element_type=jnp.float32)
    h = h + jnp.dot(x, wroot1_ref[...].T, preferred_element_type=jnp.float32)
    h = jnp.maximum(h + brel1_ref[...], 0.0)
    v1 = jnp.dot(wfc_ref[...], wrel2_ref[...], preferred_element_type=jnp.float32)
    v2 = jnp.dot(wfc_ref[...], wroot2_ref[...], preferred_element_type=jnp.float32)
    g_ref[...] = jnp.broadcast_to(jnp.sum(h * v1, axis=1, keepdims=True),
                                  (BLK, DG))
    t_ref[...] = jnp.broadcast_to(jnp.sum(h * v2, axis=1, keepdims=True),
                                  (BLK, DG))


def _mid_stage(s0, s1, x, Wrel1, brel1, Wroot1, Wrel2, Wroot2, Wfc):
    full = pl.BlockSpec((D, D), lambda i: (0, 0))
    row1 = pl.BlockSpec((1, D), lambda i: (0, 0))
    blk = pl.BlockSpec((BLK, D), lambda i: (i, 0))
    blkg = pl.BlockSpec((BLK, DG), lambda i: (i, 0))
    g16, t16 = pl.pallas_call(
        _mid_kernel,
        grid=(N // BLK,),
        in_specs=[blk, blk, blk, full, row1, full, full, full, row1],
        out_specs=[blkg, blkg],
        out_shape=[jax.ShapeDtypeStruct((N, DG), jnp.float32),
                   jax.ShapeDtypeStruct((N, DG), jnp.float32)],
    )(s0, s1, x, Wrel1, brel1.reshape(1, D), Wroot1, Wrel2, Wroot2, Wfc)
    return g16, t16


def _final_kernel(z0_ref, z1_ref, t_ref, brel2_ref, wfc_ref, bfc_ref, o_ref):
    c = jnp.sum(brel2_ref[...] * wfc_ref[...]) + bfc_ref[0, 0]
    m = z0_ref[...] + z1_ref[...] + t_ref[...]
    o_ref[...] = jnp.sum(m, axis=1) * (1.0 / DG) + c


def _final_stage(z0, z1, t16, brel2, Wfc, bfc):
    blkz = pl.BlockSpec((N, DG), lambda: (0, 0))
    row1 = pl.BlockSpec((1, D), lambda: (0, 0))
    return pl.pallas_call(
        _final_kernel,
        in_specs=[blkz, blkz, blkz, row1, row1,
                  pl.BlockSpec((1, 1), lambda: (0, 0))],
        out_specs=pl.BlockSpec((N,), lambda: (0,)),
        out_shape=jax.ShapeDtypeStruct((N,), jnp.float32),
    )(z0[:N], z1[:N], t16, brel2.reshape(1, D), Wfc, bfc.reshape(1, 1))


def kernel(x, edge_index, Wrel1, brel1, Wroot1, Wrel2, brel2, Wroot2, Wfc, bfc):
    src = edge_index[0].astype(jnp.int32)
    dst = edge_index[1].astype(jnp.int32)
    src1 = src.reshape(E // CH1, CH1)
    dst1 = dst.reshape(E // CH1, CH1)
    src3 = src.reshape(E // CH3, CH3)
    dst3 = dst.reshape(E // CH3, CH3)

    s0, s1 = _seg_rows(x, src1, dst1, jnp.zeros((NP, D), jnp.float32))
    g16, t16 = _mid_stage(s0[:N], s1[:N], x, Wrel1, brel1, Wroot1,
                          Wrel2, Wroot2, Wfc)
    z0, z1 = _seg_scalar(g16, src3, dst3, jnp.zeros((NP, DG), jnp.float32))
    return _final_stage(z0, z1, t16, brel2, Wfc, bfc)


# speedup vs baseline: 19.4971x; 1.1011x over previous
import functools

import jax
import jax.numpy as jnp
from jax import lax
from jax.experimental import pallas as pl
from jax.experimental.pallas import tpu as pltpu
from jax.experimental.pallas import tpu_sc as plsc

N = 10000
E = 320000
D = 128
DG = 16

NC = 2
NS = 16
NW = NC * NS

NP = 10240
RPT = NP // NS
EW = E // NW

_mesh = plsc.VectorSubcoreMesh(core_axis_name="c", subcore_axis_name="s")


def _make_seg_body(W, CH, NBUF, LOOK, R):
    NCH = EW // CH

    def body(tab_hbm, src_hbm, dst_hbm, zeros_hbm, out0_hbm, out1_hbm,
             src_v, dst_v, rows_v, acc_sh, *sems):
        gs, ss = sems[:NBUF], sems[NBUF:]
        cid = lax.axis_index("c")
        sid = lax.axis_index("s")
        wid = cid * NS + sid

        d0 = pltpu.async_copy(zeros_hbm.at[pl.ds(sid * RPT, RPT)],
                              acc_sh.at[pl.ds(sid * RPT, RPT)], sems[0])
        d1 = pltpu.async_copy(src_hbm.at[pl.ds(wid * NCH, NCH)], src_v,
                              sems[1])
        d2 = pltpu.async_copy(dst_hbm.at[pl.ds(wid * NCH, NCH)], dst_v,
                              sems[2])
        d0.wait()
        d1.wait()
        d2.wait()
        plsc.subcore_barrier()

        def round_body(it, carry):
            base = it * R

            def gather(s):
                b = s % NBUF
                return pltpu.async_copy(tab_hbm.at[src_v.at[base + s]],
                                        rows_v.at[b], gs[b])

            def scatter(s):
                b = s % NBUF
                return pltpu.async_copy(rows_v.at[b],
                                        acc_sh.at[dst_v.at[base + s]],
                                        ss[b], add=True)

            gd = [gather(s) for s in range(LOOK)] + [None] * (R - LOOK)
            sd = [None] * R
            for s in range(R):
                if s >= NBUF - LOOK:
                    sd[s - (NBUF - LOOK)].wait()
                if s + LOOK < R:
                    gd[s + LOOK] = gather(s + LOOK)
                gd[s].wait()
                sd[s] = scatter(s)
            for s in range(R - (NBUF - LOOK), R):
                sd[s].wait()
            return carry

        lax.fori_loop(0, NCH // R, round_body, 0, unroll=False)
        plsc.subcore_barrier()

        @pl.when(cid == 0)
        def _():
            pltpu.sync_copy(acc_sh.at[pl.ds(sid * RPT, RPT)],
                            out0_hbm.at[pl.ds(sid * RPT, RPT)])

        @pl.when(cid == 1)
        def _():
            pltpu.sync_copy(acc_sh.at[pl.ds(sid * RPT, RPT)],
                            out1_hbm.at[pl.ds(sid * RPT, RPT)])

    return body


def _make_seg_kernel(W, CH, NBUF, LOOK, R):
    NCH = EW // CH
    return functools.partial(
        pl.kernel,
        out_type=[jax.ShapeDtypeStruct((NP, W), jnp.float32),
                  jax.ShapeDtypeStruct((NP, W), jnp.float32)],
        mesh=_mesh,
        compiler_params=pltpu.CompilerParams(use_tc_tiling_on_sc=False),
        scratch_types=[
            pltpu.VMEM((NCH, CH), jnp.int32),
            pltpu.VMEM((NCH, CH), jnp.int32),
            pltpu.VMEM((NBUF, CH, W), jnp.float32),
            pltpu.VMEM_SHARED((NP, W), jnp.float32),
        ] + [pltpu.SemaphoreType.DMA] * (2 * NBUF),
    )(_make_seg_body(W, CH, NBUF, LOOK, R))


CH1 = 40
CH3 = 125

_seg_rows = _make_seg_kernel(D, CH1, 4, 2, 50)
_seg_scalar = _make_seg_kernel(DG, CH3, 8, 4, 40)


BLK = 2000


def _mid_kernel(a0_ref, a1_ref, x_ref, wrel1_ref, brel1_ref, wroot1_ref,
                wrel2_ref, wroot2_ref, wfc_ref, g_ref, t_ref):
    a = a0_ref[...] + a1_ref[...]
    x = x_ref[...]
    h = jnp.dot(a, wrel1_ref[...].T, preferred_element_type=jnp.float32)
    h = h + jnp.dot(x, wroot1_ref[...].T, preferred_element_type=jnp.float32)
    h = jnp.maximum(h + brel1_ref[...], 0.0)
    v1 = jnp.dot(wfc_ref[...], wrel2_ref[...], preferred_element_type=jnp.float32)
    v2 = jnp.dot(wfc_ref[...], wroot2_ref[...], preferred_element_type=jnp.float32)
    g_ref[...] = jnp.broadcast_to(jnp.sum(h * v1, axis=1, keepdims=True),
                                  (BLK, DG))
    t_ref[...] = jnp.broadcast_to(jnp.sum(h * v2, axis=1, keepdims=True),
                                  (BLK, DG))


def _mid_stage(s0, s1, x, Wrel1, brel1, Wroot1, Wrel2, Wroot2, Wfc):
    full = pl.BlockSpec((D, D), lambda i: (0, 0))
    row1 = pl.BlockSpec((1, D), lambda i: (0, 0))
    blk = pl.BlockSpec((BLK, D), lambda i: (i, 0))
    blkg = pl.BlockSpec((BLK, DG), lambda i: (i, 0))
    g16, t16 = pl.pallas_call(
        _mid_kernel,
        grid=(N // BLK,),
        in_specs=[blk, blk, blk, full, row1, full, full, full, row1],
        out_specs=[blkg, blkg],
        out_shape=[jax.ShapeDtypeStruct((N, DG), jnp.float32),
                   jax.ShapeDtypeStruct((N, DG), jnp.float32)],
    )(s0, s1, x, Wrel1, brel1.reshape(1, D), Wroot1, Wrel2, Wroot2, Wfc)
    return g16, t16


def _final_kernel(z0_ref, z1_ref, t_ref, brel2_ref, wfc_ref, bfc_ref, o_ref):
    c = jnp.sum(brel2_ref[...] * wfc_ref[...]) + bfc_ref[0, 0]
    m = z0_ref[...] + z1_ref[...] + t_ref[...]
    o_ref[...] = jnp.sum(m, axis=1) * (1.0 / DG) + c


def _final_stage(z0, z1, t16, brel2, Wfc, bfc):
    blkz = pl.BlockSpec((N, DG), lambda i: (0, 0))
    row1 = pl.BlockSpec((1, D), lambda i: (0, 0))
    return pl.pallas_call(
        _final_kernel,
        grid=(1,),
        in_specs=[blkz, blkz, blkz, row1, row1,
                  pl.BlockSpec((1, 1), lambda i: (0, 0))],
        out_specs=pl.BlockSpec((N,), lambda i: (0,)),
        out_shape=jax.ShapeDtypeStruct((N,), jnp.float32),
    )(z0, z1, t16, brel2.reshape(1, D), Wfc, bfc.reshape(1, 1))


def kernel(x, edge_index, Wrel1, brel1, Wroot1, Wrel2, brel2, Wroot2, Wfc, bfc):
    src = edge_index[0].astype(jnp.int32)
    dst = edge_index[1].astype(jnp.int32)
    src1 = src.reshape(E // CH1, CH1)
    dst1 = dst.reshape(E // CH1, CH1)
    src3 = src.reshape(E // CH3, CH3)
    dst3 = dst.reshape(E // CH3, CH3)

    s0, s1 = _seg_rows(x, src1, dst1, jnp.zeros((NP, D), jnp.float32))
    g16, t16 = _mid_stage(s0, s1, x, Wrel1, brel1, Wroot1,
                          Wrel2, Wroot2, Wfc)
    z0, z1 = _seg_scalar(g16, src3, dst3, jnp.zeros((NP, DG), jnp.float32))
    return _final_stage(z0, z1, t16, brel2, Wfc, bfc)
